# baseline (speedup 1.0000x reference)
import functools

import jax
import jax.numpy as jnp
from jax import lax
from jax.experimental import pallas as pl
from jax.experimental.pallas import tpu as pltpu
from jax.experimental.pallas import tpu_sc as plsc

N = 4096
HID = 256
E = 262144

NC = 2
NS = 16
LANES = 16

EPT = E // NS
CH = 2048
NCH = EPT // CH
NBLK = 8
RB = N // (NC * NBLK)
BLK_W = RB * N
TPW = BLK_W // NS
ZB = 4096


def _tc_scores(emb, w_pad, b_pad):
    def body(emb_ref, w_ref, b_ref, out_ref):
        out_ref[...] = (
            jnp.dot(emb_ref[...], w_ref[...], preferred_element_type=jnp.float32)
            + b_ref[...]
        )

    return pl.pallas_call(
        body,
        out_shape=jax.ShapeDtypeStruct((N, 128), jnp.float32),
    )(emb, w_pad, b_pad)


def _sc_scatter(s1, s2, v0, v1):
    mesh = plsc.VectorSubcoreMesh(core_axis_name="c", subcore_axis_name="s")

    @functools.partial(
        pl.kernel,
        out_type=jax.ShapeDtypeStruct((N * N,), jnp.float32),
        mesh=mesh,
        scratch_types=[
            pltpu.VMEM((CH,), jnp.int32),
            pltpu.VMEM((CH,), jnp.int32),
            pltpu.VMEM((CH,), jnp.int32),
            pltpu.VMEM((CH,), jnp.int32),
            pltpu.VMEM((CH,), jnp.float32),
            pltpu.VMEM((CH,), jnp.float32),
            pltpu.VMEM((CH,), jnp.float32),
            pltpu.VMEM((CH,), jnp.float32),
            pltpu.VMEM((EPT + 16,), jnp.int32),
            pltpu.VMEM((EPT + 16,), jnp.float32),
            pltpu.VMEM((CH,), jnp.int32),
            pltpu.VMEM((CH,), jnp.int32),
            pltpu.VMEM((CH,), jnp.float32),
            pltpu.VMEM((CH,), jnp.float32),
            pltpu.VMEM((ZB,), jnp.float32),
            pltpu.VMEM_SHARED((BLK_W,), jnp.float32),
            pltpu.SemaphoreType.DMA,
            pltpu.SemaphoreType.DMA,
            pltpu.SemaphoreType.DMA,
            pltpu.SemaphoreType.DMA,
            pltpu.SemaphoreType.DMA,
            pltpu.SemaphoreType.DMA,
            pltpu.SemaphoreType.DMA,
            pltpu.SemaphoreType.DMA,
            pltpu.SemaphoreType.DMA,
            pltpu.SemaphoreType.DMA,
            pltpu.SemaphoreType.DMA,
        ],
    )
    def k(s1_hbm, s2_hbm, v0_hbm, v1_hbm, z_hbm,
          vi0a, vi0b, vi1a, vi1b, x1a, x1b, x2a, x2b, g_v, val_v,
          sia, sib, sva, svb, zb, acc,
          sem_v0, sem_v1, sem_x1, sem_x2, sem_s0, sem_s1, sem_z,
          sem_d0, sem_d1, sem_d2, sem_d3):
        cid = lax.axis_index("c")
        sid = lax.axis_index("s")
        sem_s = [sem_s0, sem_s1]
        sem_d = [sem_d0, sem_d1, sem_d2, sem_d3]
        vi0c = [vi0a, vi0b]
        vi1c = [vi1a, vi1b]
        x1c = [x1a, x1b]
        x2c = [x2a, x2b]
        st_i = [sia, sib]
        st_v = [sva, svb]

        zvec = jnp.zeros((LANES,), jnp.float32)

        def zero_body(i, _):
            zb[pl.ds(i * LANES, LANES)] = zvec
            return 0

        lax.fori_loop(0, ZB // LANES, zero_body, 0)

        def vstart(ci):
            s = ci % 2
            ebase = sid * EPT + ci * CH
            h0 = pltpu.async_copy(v0_hbm.at[pl.ds(ebase, CH)], vi0c[s], sem_v0)
            h1 = pltpu.async_copy(v1_hbm.at[pl.ds(ebase, CH)], vi1c[s], sem_v1)
            return (h0, h1)

        def compute(ci):
            s = ci % 2

            def pre_body(i, _):
                a = vi0c[s][pl.ds(i * LANES, LANES)]
                c = vi1c[s][pl.ds(i * LANES, LANES)]
                g_v[pl.ds(ci * CH + i * LANES, LANES)] = (a << 12) + c
                return 0

            lax.fori_loop(0, CH // LANES, pre_body, 0)

        vh = {0: vstart(0)}
        for ci in range(NCH):
            for h in vh.pop(ci):
                h.wait()
            if ci + 1 < NCH:
                vh[ci + 1] = vstart(ci + 1)
            compute(ci)

        npair = EPT // (2 * CH)

        def pair2_body(pi, _):
            for s in range(2):
                poff = pi * (2 * CH) + s * CH

                def idx_body(i, _):
                    sg = g_v[pl.ds(poff + i * LANES, LANES)]
                    vi0c[s][pl.ds(i * LANES, LANES)] = (sg >> 12) & (N - 1)
                    vi1c[s][pl.ds(i * LANES, LANES)] = sg & (N - 1)
                    return 0

                lax.fori_loop(0, CH // LANES, idx_body, 0)
                pltpu.async_copy(s1_hbm.at[vi0c[s]], x1c[s], sem_x1)
                pltpu.async_copy(s2_hbm.at[vi1c[s]], x2c[s], sem_x2)

            for s in range(2):
                poff = pi * (2 * CH) + s * CH
                pltpu.make_async_copy(s1_hbm.at[vi0c[s]], x1c[s], sem_x1).wait()
                pltpu.make_async_copy(s2_hbm.at[vi1c[s]], x2c[s], sem_x2).wait()

                def val_body(i, _):
                    val_v[pl.ds(poff + i * LANES, LANES)] = (
                        x1c[s][pl.ds(i * LANES, LANES)]
                        + x2c[s][pl.ds(i * LANES, LANES)]
                    )
                    return 0

                lax.fori_loop(0, CH // LANES, val_body, 0)
            return 0

        lax.fori_loop(0, npair, pair2_body, 0)

        zh = []
        for j in range(TPW // ZB):
            zh.append(
                pltpu.async_copy(zb, acc.at[pl.ds(sid * TPW + j * ZB, ZB)], sem_z)
            )
        for h in zh:
            h.wait()
        plsc.subcore_barrier()

        for blk in range(NBLK):
            base = (cid * NBLK + blk) * BLK_W

            def pair_body(pi, _):
                @pl.when(pi > 0)
                def _():
                    for s in range(2):
                        pltpu.make_async_copy(
                            st_v[s], acc.at[st_i[s]], sem_s[s]
                        ).wait()

                for s in range(2):
                    poff = pi * (2 * CH) + s * CH

                    def vec_body(i, _):
                        idx16 = g_v[pl.ds(poff + i * LANES, LANES)] - base
                        v16 = val_v[pl.ds(poff + i * LANES, LANES)]
                        inr = plsc.bitcast(idx16, jnp.uint32) < jnp.uint32(BLK_W)
                        st_i[s][pl.ds(i * LANES, LANES)] = idx16 & (BLK_W - 1)
                        st_v[s][pl.ds(i * LANES, LANES)] = jnp.where(inr, v16, 0.0)
                        return 0

                    lax.fori_loop(0, CH // LANES, vec_body, 0)
                    pltpu.async_copy(
                        st_v[s], acc.at[st_i[s]], sem_s[s], add=True
                    )
                return 0

            lax.fori_loop(0, npair, pair_body, 0)

            for s in range(2):
                pltpu.make_async_copy(
                    st_v[s], acc.at[st_i[s]], sem_s[s]
                ).wait()

            plsc.subcore_barrier()

            PW = TPW // 4
            dh = []
            for j in range(4):
                dh.append(
                    pltpu.async_copy(
                        acc.at[pl.ds(sid * TPW + j * PW, PW)],
                        z_hbm.at[pl.ds(base + sid * TPW + j * PW, PW)],
                        sem_d[j],
                    )
                )
            if blk + 1 < NBLK:
                zh = []
                for j in range(4):
                    dh[j].wait()
                    for q in range(PW // ZB):
                        zh.append(
                            pltpu.async_copy(
                                zb,
                                acc.at[pl.ds(sid * TPW + j * PW + q * ZB, ZB)],
                                sem_z,
                            )
                        )
                for h in zh:
                    h.wait()
                plsc.subcore_barrier()
            else:
                for h in dh:
                    h.wait()

    return k(s1, s2, v0, v1)


def kernel(embeddings, v_indices, W, b):
    w_pad = jnp.zeros((HID, 128), jnp.float32)
    w_pad = w_pad.at[:, 0].set(W[0, :HID])
    w_pad = w_pad.at[:, 1].set(W[0, HID:])
    b_pad = jnp.zeros((1, 128), jnp.float32).at[0, 0].set(b[0])

    scores = _tc_scores(embeddings, w_pad, b_pad)
    s1 = scores[:, 0]
    s2 = scores[:, 1]

    zflat = _sc_scatter(s1, s2, v_indices[0], v_indices[1])
    return zflat.reshape(N, N)
